# baseline (speedup 1.0000x reference)
import functools

import jax
import jax.numpy as jnp
from jax import lax
from jax.experimental import pallas as pl
from jax.experimental.pallas import tpu as pltpu
from jax.experimental.pallas import tpu_sc as plsc

NC, NS = 2, 16
NW = NC * NS
GATHER = 128
TB = 2048


def _tr_body(in_b, out):
    t = jnp.transpose(in_b[...], (1, 0))
    out[:, 0:64] = t
    out[:, 64:128] = t


def _pack_table(wT, V):
    grid = (V + TB - 1) // TB
    return pl.pallas_call(
        _tr_body,
        grid=(grid,),
        in_specs=[pl.BlockSpec((64, TB), lambda i: (0, i))],
        out_specs=pl.BlockSpec((TB, 128), lambda i: (i, 0)),
        out_shape=jax.ShapeDtypeStruct((V, 128), jnp.float32),
    )(wT)


def _gbody(w2, rows, o3, rows_v, S, T, gsem, ssem):
    wid = lax.axis_index("s") * NC + lax.axis_index("c")
    H, D, B = o3.shape
    b_per_w = B // NW
    nblk = b_per_w // GATHER
    chunks = H * nblk
    base_b = wid * b_per_w

    pltpu.sync_copy(rows.at[wid], rows_v)

    iotas = [
        lax.broadcasted_iota(jnp.int32, (16,), 0) + i0 * 16 for i0 in range(8)
    ]
    zeros16 = jnp.zeros((16,), jnp.int32)

    def fire_gather(c, b):
        pltpu.async_copy(w2.at[rows_v.at[c]], S.at[b], gsem.at[b])

    def drain_gather(b):
        pltpu.make_async_copy(
            w2.at[pl.ds(0, GATHER)], S.at[b], gsem.at[b]
        ).wait()

    def transpose_block(b):
        Sb = S.at[b]
        Tb = T.at[b]
        for d in range(D):
            col = zeros16 + d
            for i0 in range(8):
                vals = plsc.load_gather(Sb, [iotas[i0], col])
                Tb[d, pl.ds(i0 * 16, 16)] = vals

    def fire_store(c, b):
        h = c // nblk
        b0 = base_b + (c % nblk) * GATHER
        pltpu.async_copy(T.at[b], o3.at[h, :, pl.ds(b0, GATHER)], ssem.at[b])

    def drain_store(b):
        pltpu.make_async_copy(
            T.at[b], o3.at[0, :, pl.ds(0, GATHER)], ssem.at[b]
        ).wait()

    fire_gather(0, 0)

    @pl.loop(0, chunks // 2)
    def _pipe(g):
        for j in range(2):
            c = 2 * g + j
            @pl.when(c + 1 < chunks)
            def _():
                fire_gather(c + 1, 1 - j)
            drain_gather(j)
            @pl.when(g >= 1)
            def _():
                drain_store(j)
            transpose_block(j)
            fire_store(c, j)

    drain_store(0)
    drain_store(1)


def kernel(x, weight):
    B, H = x.shape
    V, D = weight.shape

    w2 = _pack_table(weight.T, V)

    nblk = B // NW // GATHER
    chunks = H * nblk
    xr = x.reshape(NW, nblk, GATHER, H).astype(jnp.int32)
    rows = xr.transpose(0, 3, 1, 2).reshape(NW, chunks, GATHER)

    mesh = plsc.VectorSubcoreMesh(
        core_axis_name="c", subcore_axis_name="s",
        num_cores=NC, num_subcores=NS,
    )
    run = pl.kernel(
        _gbody,
        out_type=jax.ShapeDtypeStruct((H, D, B), jnp.float32),
        mesh=mesh,
        scratch_types=[
            pltpu.VMEM((chunks, GATHER), jnp.int32),
            pltpu.VMEM((2, GATHER, 2 * D), jnp.float32),
            pltpu.VMEM((2, D, GATHER), jnp.float32),
            pltpu.SemaphoreType.DMA((2,)),
            pltpu.SemaphoreType.DMA((2,)),
        ],
        compiler_params=pltpu.CompilerParams(
            use_tc_tiling_on_sc=True, needs_layout_passes=False
        ),
    )
    o3 = run(w2, rows)
    return o3.transpose(2, 0, 1)


# speedup vs baseline: 1.5304x; 1.5304x over previous
import functools

import jax
import jax.numpy as jnp
from jax import lax
from jax.experimental import pallas as pl
from jax.experimental.pallas import tpu as pltpu
from jax.experimental.pallas import tpu_sc as plsc

NC, NS = 2, 16
NW = NC * NS
GATHER = 128
K = 4
CHUNK = K * GATHER


def _body(table, idx, out, idx_v, rows_v, gsem, ssem):
    wid = lax.axis_index("s") * NC + lax.axis_index("c")
    rows_total = out.shape[0]
    b_per_w = rows_total // NW
    chunks = b_per_w // CHUNK
    base_row = wid * b_per_w

    pltpu.sync_copy(idx.at[pl.ds(wid * chunks, chunks)], idx_v)

    def fire_gathers(c, b):
        for j in range(K):
            pltpu.async_copy(
                table.at[idx_v.at[c, j]],
                rows_v.at[b, pl.ds(j * GATHER, GATHER)],
                gsem.at[b],
            )

    def drain_gathers(b):
        pltpu.make_async_copy(
            out.at[pl.ds(0, CHUNK)], rows_v.at[b], gsem.at[b]
        ).wait()

    def fire_store(c, b):
        pltpu.async_copy(
            rows_v.at[b],
            out.at[pl.ds(base_row + c * CHUNK, CHUNK)],
            ssem.at[b],
        )

    def drain_store(b):
        pltpu.make_async_copy(
            rows_v.at[b], out.at[pl.ds(0, CHUNK)], ssem.at[b]
        ).wait()

    fire_gathers(0, 0)

    @pl.loop(0, chunks - 1)
    def _pipe(c):
        b = c % 2
        nb = 1 - b
        @pl.when(c >= 1)
        def _():
            drain_store(nb)
        fire_gathers(c + 1, nb)
        drain_gathers(b)
        fire_store(c, b)

    last = chunks - 1
    lb = last % 2
    drain_gathers(lb)
    fire_store(last, lb)
    drain_store(lb)
    drain_store(1 - lb)


def kernel(x, weight):
    B, H = x.shape
    V, D = weight.shape
    rows = B * H
    idx3d = x.reshape(rows // CHUNK, K, GATHER).astype(jnp.int32)
    chunks_per_w = rows // NW // CHUNK

    mesh = plsc.VectorSubcoreMesh(
        core_axis_name="c", subcore_axis_name="s",
        num_cores=NC, num_subcores=NS,
    )
    run = pl.kernel(
        _body,
        out_type=jax.ShapeDtypeStruct((rows, D), jnp.float32),
        mesh=mesh,
        scratch_types=[
            pltpu.VMEM((chunks_per_w, K, GATHER), jnp.int32),
            pltpu.VMEM((2, CHUNK, D), jnp.float32),
            pltpu.SemaphoreType.DMA((2,)),
            pltpu.SemaphoreType.DMA((2,)),
        ],
        compiler_params=pltpu.CompilerParams(use_tc_tiling_on_sc=False),
    )
    out = run(weight, idx3d)
    return out.reshape(B, H, D)
